# baseline (speedup 1.0000x reference)
import functools

import jax
import jax.numpy as jnp
from jax.experimental import pallas as pl
from jax.experimental.pallas import tpu as pltpu

_BF = jnp.bfloat16

_TAP = {0: (1, -1), 1: (0, 0), 2: (1, 0)}


def _shift(g, sr, sc, pad_val=0.0):
    if sr:
        H = g.shape[1]
        g = jnp.pad(g, ((0, 0), (1, 0), (0, 0), (0, 0)),
                    constant_values=pad_val)[:, :H]
    if sc:
        W = g.shape[2]
        g = jnp.pad(g, ((0, 0), (0, 0), (1, 0), (0, 0)),
                    constant_values=pad_val)[:, :, :W]
    return g


def _stem_kernel(pee_ref, peo_ref, poe_ref, poo_ref, w_ref, o_ref):
    w = w_ref[...]
    grids = []
    for pref in (pee_ref, peo_ref, poe_ref, poo_ref):
        a = pref[...].reshape(56 * 56, 147)
        y = jnp.dot(a, w, preferred_element_type=jnp.float32)
        grids.append(jnp.maximum(y, 0.0).astype(_BF).reshape(1, 56, 56, 64))
    yg = [[grids[0], grids[1]], [grids[2], grids[3]]]
    ninf = float(-jnp.inf)
    m = None
    for di in range(3):
        pr, sr = _TAP[di]
        for dj in range(3):
            pc, sc = _TAP[dj]
            t = _shift(yg[pr][pc], sr, sc, pad_val=ninf)
            m = t if m is None else jnp.maximum(m, t)
    o_ref[...] = m


def _stem(x_nchw, conv1_w):
    xh = jnp.transpose(x_nchw, (0, 2, 3, 1)).astype(_BF)
    B = xh.shape[0]
    xp = jnp.pad(xh, ((0, 0), (3, 3), (3, 3), (0, 0)))
    pats = []
    for a in range(2):
        for b in range(2):
            cols = [xp[:, 2 * a + ki:2 * a + ki + 221:4,
                       2 * b + kj:2 * b + kj + 221:4, :]
                    for ki in range(7) for kj in range(7)]
            pats.append(jnp.concatenate(cols, axis=-1))
    spec = pl.BlockSpec((1, 56, 56, 147), lambda i: (i, 0, 0, 0))
    return pl.pallas_call(
        _stem_kernel,
        out_shape=jax.ShapeDtypeStruct((B, 56, 56, 64), _BF),
        grid=(B,),
        in_specs=[spec, spec, spec, spec,
                  pl.BlockSpec((147, 64), lambda i: (0, 0))],
        out_specs=pl.BlockSpec((1, 56, 56, 64), lambda i: (i, 0, 0, 0)),
        compiler_params=pltpu.CompilerParams(
            dimension_semantics=("parallel",)),
    )(*pats, conv1_w)


def _bneck_kernel(*refs, bb, H, W, Cmid, has_ds):
    if has_ds:
        x_ref, w1_ref, w2_ref, w3_ref, wds_ref, o_ref = refs
    else:
        x_ref, w1_ref, w2_ref, w3_ref, o_ref = refs
    Cin = x_ref.shape[-1]
    Cout = o_ref.shape[-1]
    x2d = x_ref[...].reshape(bb * H * W, Cin)
    y1 = jnp.dot(x2d, w1_ref[...], preferred_element_type=jnp.float32)
    y1 = jnp.maximum(y1, 0.0).astype(_BF).reshape(bb, H, W, Cmid)
    y1p = jnp.pad(y1, ((0, 0), (1, 1), (1, 1), (0, 0)))
    acc = jnp.zeros((bb * H * W, Cmid), jnp.float32)
    for di in range(3):
        for dj in range(3):
            t = y1p[:, di:di + H, dj:dj + W, :]
            wt = w2_ref[(di * 3 + dj) * Cmid:(di * 3 + dj + 1) * Cmid, :]
            acc = acc + jnp.dot(t.reshape(bb * H * W, Cmid), wt,
                                preferred_element_type=jnp.float32)
    y2 = jnp.maximum(acc, 0.0).astype(_BF)
    y3 = jnp.dot(y2, w3_ref[...], preferred_element_type=jnp.float32)
    if has_ds:
        ident = jnp.dot(x2d, wds_ref[...],
                        preferred_element_type=jnp.float32).astype(_BF)
    else:
        ident = x2d
    r = y3 + ident.astype(jnp.float32)
    o_ref[...] = jnp.maximum(r, 0.0).astype(_BF).reshape(bb, H, W, Cout)


def _fused_bottleneck(x, w1, w2, w3, wds=None, bb=1):
    B, H, W, Cin = x.shape
    Cmid = w1.shape[1]
    Cout = w3.shape[1]
    kern = functools.partial(_bneck_kernel, bb=bb, H=H, W=W, Cmid=Cmid,
                             has_ds=wds is not None)
    in_specs = [
        pl.BlockSpec((bb, H, W, Cin), lambda i: (i, 0, 0, 0)),
        pl.BlockSpec((Cin, Cmid), lambda i: (0, 0)),
        pl.BlockSpec((9 * Cmid, Cmid), lambda i: (0, 0)),
        pl.BlockSpec((Cmid, Cout), lambda i: (0, 0)),
    ]
    args = [x, w1, w2, w3]
    if wds is not None:
        in_specs.append(pl.BlockSpec((Cin, Cout), lambda i: (0, 0)))
        args.append(wds)
    return pl.pallas_call(
        kern,
        out_shape=jax.ShapeDtypeStruct((B, H, W, Cout), _BF),
        grid=(B // bb,),
        in_specs=in_specs,
        out_specs=pl.BlockSpec((bb, H, W, Cout), lambda i: (i, 0, 0, 0)),
        compiler_params=pltpu.CompilerParams(
            dimension_semantics=("parallel",)),
    )(*args)


def _tbneck_kernel(xee_ref, xeo_ref, xoe_ref, xoo_ref,
                   w1_ref, w2_ref, w3_ref, wds_ref, o_ref, *, bb, Ho, Wo,
                   Cmid):
    Cin = xee_ref.shape[-1]
    Cout = o_ref.shape[-1]
    M = bb * Ho * Wo
    w1 = w1_ref[...]
    yg = [[None, None], [None, None]]
    for idx, xref in enumerate((xee_ref, xeo_ref, xoe_ref, xoo_ref)):
        a = xref[...].reshape(M, Cin)
        y = jnp.dot(a, w1, preferred_element_type=jnp.float32)
        yg[idx // 2][idx % 2] = jnp.maximum(y, 0.0).astype(_BF).reshape(
            bb, Ho, Wo, Cmid)
    acc = jnp.zeros((M, Cmid), jnp.float32)
    for di in range(3):
        pr, sr = _TAP[di]
        for dj in range(3):
            pc, sc = _TAP[dj]
            t = _shift(yg[pr][pc], sr, sc)
            wt = w2_ref[(di * 3 + dj) * Cmid:(di * 3 + dj + 1) * Cmid, :]
            acc = acc + jnp.dot(t.reshape(M, Cmid), wt,
                                preferred_element_type=jnp.float32)
    y2 = jnp.maximum(acc, 0.0).astype(_BF)
    y3 = jnp.dot(y2, w3_ref[...], preferred_element_type=jnp.float32)
    xs = xee_ref[...].reshape(M, Cin)
    ident = jnp.dot(xs, wds_ref[...],
                    preferred_element_type=jnp.float32).astype(_BF)
    r = y3 + ident.astype(jnp.float32)
    o_ref[...] = jnp.maximum(r, 0.0).astype(_BF).reshape(bb, Ho, Wo, Cout)


def _fused_transition(x, w1, w2, w3, wds, bb=1):
    B, H, W, Cin = x.shape
    Cmid = w1.shape[1]
    Cout = w3.shape[1]
    Ho, Wo = H // 2, W // 2
    xg = [x[:, a::2, b::2, :] for a in range(2) for b in range(2)]
    kern = functools.partial(_tbneck_kernel, bb=bb, Ho=Ho, Wo=Wo, Cmid=Cmid)
    xspec = pl.BlockSpec((bb, Ho, Wo, Cin), lambda i: (i, 0, 0, 0))
    return pl.pallas_call(
        kern,
        out_shape=jax.ShapeDtypeStruct((B, Ho, Wo, Cout), _BF),
        grid=(B // bb,),
        in_specs=[xspec, xspec, xspec, xspec,
                  pl.BlockSpec((Cin, Cmid), lambda i: (0, 0)),
                  pl.BlockSpec((9 * Cmid, Cmid), lambda i: (0, 0)),
                  pl.BlockSpec((Cmid, Cout), lambda i: (0, 0)),
                  pl.BlockSpec((Cin, Cout), lambda i: (0, 0))],
        out_specs=pl.BlockSpec((bb, Ho, Wo, Cout), lambda i: (i, 0, 0, 0)),
        compiler_params=pltpu.CompilerParams(
            dimension_semantics=("parallel",)),
    )(*xg, w1, w2, w3, wds)


def _head_kernel(x_ref, w_ref, b_ref, o_ref):
    feat = jnp.mean(x_ref[...].astype(jnp.float32), axis=1).astype(_BF)
    o_ref[...] = jnp.dot(feat, w_ref[...],
                         preferred_element_type=jnp.float32) + b_ref[...]


def _head(y, fc_w, fc_b):
    B = y.shape[0]
    feats = y.reshape(B, 49, 2048)
    bb = 16 if B % 16 == 0 else B
    return pl.pallas_call(
        _head_kernel,
        out_shape=jax.ShapeDtypeStruct((B, 120), jnp.float32),
        grid=(B // bb,),
        in_specs=[
            pl.BlockSpec((bb, 49, 2048), lambda i: (i, 0, 0)),
            pl.BlockSpec((2048, 120), lambda i: (0, 0)),
            pl.BlockSpec((1, 120), lambda i: (0, 0)),
        ],
        out_specs=pl.BlockSpec((bb, 120), lambda i: (i, 0)),
        compiler_params=pltpu.CompilerParams(
            dimension_semantics=("parallel",)),
    )(feats, fc_w, fc_b.astype(jnp.float32))


def kernel(x, conv1_w, b0_c1_w, b0_c2_w, b0_c3_w, b0_ds_w, b1_c1_w, b1_c2_w, b1_c3_w, b2_c1_w, b2_c2_w, b2_c3_w, b3_c1_w, b3_c2_w, b3_c3_w, b3_ds_w, b4_c1_w, b4_c2_w, b4_c3_w, b5_c1_w, b5_c2_w, b5_c3_w, b6_c1_w, b6_c2_w, b6_c3_w, b7_c1_w, b7_c2_w, b7_c3_w, b8_c1_w, b8_c2_w, b8_c3_w, b9_c1_w, b9_c2_w, b9_c3_w, b10_c1_w, b10_c2_w, b10_c3_w, b11_c1_w, b11_c2_w, b11_c3_w, b11_ds_w, b12_c1_w, b12_c2_w, b12_c3_w, b13_c1_w, b13_c2_w, b13_c3_w, b14_c1_w, b14_c2_w, b14_c3_w, b15_c1_w, b15_c2_w, b15_c3_w, b16_c1_w, b16_c2_w, b16_c3_w, b17_c1_w, b17_c2_w, b17_c3_w, b18_c1_w, b18_c2_w, b18_c3_w, b19_c1_w, b19_c2_w, b19_c3_w, b20_c1_w, b20_c2_w, b20_c3_w, b21_c1_w, b21_c2_w, b21_c3_w, b22_c1_w, b22_c2_w, b22_c3_w, b23_c1_w, b23_c2_w, b23_c3_w, b24_c1_w, b24_c2_w, b24_c3_w, b25_c1_w, b25_c2_w, b25_c3_w, b26_c1_w, b26_c2_w, b26_c3_w, b27_c1_w, b27_c2_w, b27_c3_w, b28_c1_w, b28_c2_w, b28_c3_w, b29_c1_w, b29_c2_w, b29_c3_w, b30_c1_w, b30_c2_w, b30_c3_w, b31_c1_w, b31_c2_w, b31_c3_w, b32_c1_w, b32_c2_w, b32_c3_w, b33_c1_w, b33_c2_w, b33_c3_w, b34_c1_w, b34_c2_w, b34_c3_w, b35_c1_w, b35_c2_w, b35_c3_w, b36_c1_w, b36_c2_w, b36_c3_w, b37_c1_w, b37_c2_w, b37_c3_w, b38_c1_w, b38_c2_w, b38_c3_w, b39_c1_w, b39_c2_w, b39_c3_w, b40_c1_w, b40_c2_w, b40_c3_w, b41_c1_w, b41_c2_w, b41_c3_w, b42_c1_w, b42_c2_w, b42_c3_w, b43_c1_w, b43_c2_w, b43_c3_w, b44_c1_w, b44_c2_w, b44_c3_w, b45_c1_w, b45_c2_w, b45_c3_w, b46_c1_w, b46_c2_w, b46_c3_w, b47_c1_w, b47_c2_w, b47_c3_w, b47_ds_w, b48_c1_w, b48_c2_w, b48_c3_w, b49_c1_w, b49_c2_w, b49_c3_w, fc_w, fc_b):
    c1 = [b0_c1_w, b1_c1_w, b2_c1_w, b3_c1_w, b4_c1_w, b5_c1_w, b6_c1_w,
          b7_c1_w, b8_c1_w, b9_c1_w, b10_c1_w, b11_c1_w, b12_c1_w, b13_c1_w,
          b14_c1_w, b15_c1_w, b16_c1_w, b17_c1_w, b18_c1_w, b19_c1_w,
          b20_c1_w, b21_c1_w, b22_c1_w, b23_c1_w, b24_c1_w, b25_c1_w,
          b26_c1_w, b27_c1_w, b28_c1_w, b29_c1_w, b30_c1_w, b31_c1_w,
          b32_c1_w, b33_c1_w, b34_c1_w, b35_c1_w, b36_c1_w, b37_c1_w,
          b38_c1_w, b39_c1_w, b40_c1_w, b41_c1_w, b42_c1_w, b43_c1_w,
          b44_c1_w, b45_c1_w, b46_c1_w, b47_c1_w, b48_c1_w, b49_c1_w]
    c2 = [b0_c2_w, b1_c2_w, b2_c2_w, b3_c2_w, b4_c2_w, b5_c2_w, b6_c2_w,
          b7_c2_w, b8_c2_w, b9_c2_w, b10_c2_w, b11_c2_w, b12_c2_w, b13_c2_w,
          b14_c2_w, b15_c2_w, b16_c2_w, b17_c2_w, b18_c2_w, b19_c2_w,
          b20_c2_w, b21_c2_w, b22_c2_w, b23_c2_w, b24_c2_w, b25_c2_w,
          b26_c2_w, b27_c2_w, b28_c2_w, b29_c2_w, b30_c2_w, b31_c2_w,
          b32_c2_w, b33_c2_w, b34_c2_w, b35_c2_w, b36_c2_w, b37_c2_w,
          b38_c2_w, b39_c2_w, b40_c2_w, b41_c2_w, b42_c2_w, b43_c2_w,
          b44_c2_w, b45_c2_w, b46_c2_w, b47_c2_w, b48_c2_w, b49_c2_w]
    c3 = [b0_c3_w, b1_c3_w, b2_c3_w, b3_c3_w, b4_c3_w, b5_c3_w, b6_c3_w,
          b7_c3_w, b8_c3_w, b9_c3_w, b10_c3_w, b11_c3_w, b12_c3_w, b13_c3_w,
          b14_c3_w, b15_c3_w, b16_c3_w, b17_c3_w, b18_c3_w, b19_c3_w,
          b20_c3_w, b21_c3_w, b22_c3_w, b23_c3_w, b24_c3_w, b25_c3_w,
          b26_c3_w, b27_c3_w, b28_c3_w, b29_c3_w, b30_c3_w, b31_c3_w,
          b32_c3_w, b33_c3_w, b34_c3_w, b35_c3_w, b36_c3_w, b37_c3_w,
          b38_c3_w, b39_c3_w, b40_c3_w, b41_c3_w, b42_c3_w, b43_c3_w,
          b44_c3_w, b45_c3_w, b46_c3_w, b47_c3_w, b48_c3_w, b49_c3_w]
    ds = {0: b0_ds_w, 3: b3_ds_w, 11: b11_ds_w, 47: b47_ds_w}
    tbb = {3: 1, 11: 2, 47: 4}

    y = _stem(x, conv1_w)
    for i in range(50):
        if i in tbb:
            y = _fused_transition(y, c1[i], c2[i], c3[i], ds[i], bb=tbb[i])
        else:
            H = y.shape[1]
            bb = 1 if H >= 28 else (4 if H == 14 else 8)
            y = _fused_bottleneck(y, c1[i], c2[i], c3[i], wds=ds.get(i),
                                  bb=bb)
    return _head(y, fc_w, fc_b)
